# baseline (speedup 1.0000x reference)
import functools

import jax
import jax.numpy as jnp
from jax import lax
from jax.experimental import pallas as pl
from jax.experimental.pallas import tpu as pltpu
from jax.experimental.pallas import tpu_sc as plsc

S = 64.0
MEAN = 0.35
SIGMA = 0.0125

N_ROWS = 1024
N_COLS = 100000
NC, NS, L = 2, 16, 16
NW = NC * NS
RB = 16
NB = N_COLS // RB
NIT = (NB + NW - 1) // NW
NSLOT = 4
NQ = N_ROWS // L

_mesh = plsc.VectorSubcoreMesh(
    core_axis_name="c", subcore_axis_name="s", num_cores=NC, num_subcores=NS
)


@functools.partial(
    pl.kernel,
    out_type=jax.ShapeDtypeStruct((N_COLS, N_ROWS), jnp.float32),
    mesh=_mesh,
    scratch_types=[
        pltpu.VMEM((N_ROWS,), jnp.int32),
        pltpu.VMEM((N_ROWS,), jnp.float32),
        pltpu.VMEM((NSLOT, RB, N_ROWS), jnp.float32),
        pltpu.SemaphoreType.DMA,
        pltpu.SemaphoreType.DMA,
    ],
    compiler_params=pltpu.CompilerParams(needs_layout_passes=False),
)
def _sc_scale_t(lt_hbm, labels_hbm, ela_hbm, out_hbm, lab_v, ela_v, buf,
                in_sem, out_sem):
    wid = lax.axis_index("s") * NC + lax.axis_index("c")
    pltpu.sync_copy(labels_hbm, lab_v)
    pltpu.sync_copy(ela_hbm, ela_v)
    lanes = lax.iota(jnp.int32, L)

    def block_of(t):
        return t * NW + wid

    def valid(t):
        return block_of(t) < NB

    def row0_of(t):
        return pl.multiple_of(block_of(t) * RB, RB)

    def src(t):
        return lt_hbm.at[pl.ds(row0_of(t), RB), :]

    def dst(t):
        return out_hbm.at[pl.ds(row0_of(t), RB), :]

    @pl.when(valid(0))
    def _():
        pltpu.async_copy(src(0), buf.at[0], in_sem)

    @pl.when(valid(1))
    def _():
        pltpu.async_copy(src(1), buf.at[1], in_sem)

    def step(t, k):
        kn = (k + 2) % NSLOT

        @pl.when((t >= 2) & valid(jnp.maximum(t - 2, 0)))
        def _():
            tp = jnp.maximum(t - 2, 0)
            pltpu.make_async_copy(buf.at[kn], dst(tp), out_sem).wait()

        @pl.when((t + 2 < NIT) & valid(jnp.minimum(t + 2, NIT - 1)))
        def _():
            tn = jnp.minimum(t + 2, NIT - 1)
            pltpu.async_copy(src(tn), buf.at[kn], in_sem)

        @pl.when(valid(t))
        def _():
            pltpu.make_async_copy(src(t), buf.at[k], in_sem).wait()

            def scale_loop(q, carry):
                o = q * L
                for r in range(RB):
                    buf[k, r, pl.ds(o, L)] = buf[k, r, pl.ds(o, L)] * S
                return carry

            lax.fori_loop(0, NQ, scale_loop, 0, unroll=2)

            row0 = block_of(t) * RB

            def margin_loop(q, carry):
                o = pl.multiple_of(q * L, L)
                lab16 = lab_v[pl.ds(o, L)]
                ela16 = ela_v[pl.ds(o, L)]
                off = lab16 - row0
                m = (off >= 0) & (off < RB)
                offc = jnp.clip(off, 0, RB - 1)
                cols16 = o + lanes
                slot = buf.at[k]
                v = plsc.load_gather(slot, [offc, cols16], mask=m)
                plsc.store_scatter(slot, [offc, cols16], v - ela16, mask=m)
                return carry

            lax.fori_loop(0, NQ, margin_loop, 0)

            pltpu.async_copy(buf.at[k], dst(t), out_sem)

    def ring_loop(tt, carry):
        for kk in range(NSLOT):
            step(tt * NSLOT + kk, kk)
        return carry

    lax.fori_loop(0, NIT // NSLOT, ring_loop, 0)

    for back in (2, 1):
        tl = NIT - back

        @pl.when(valid(tl))
        def _():
            pltpu.make_async_copy(buf.at[tl % NSLOT], dst(tl), out_sem).wait()


def kernel(logits, labels):
    ekey = jax.random.key(42)
    ela_s = (MEAN + SIGMA * jax.random.normal(ekey, (N_ROWS,), dtype=jnp.float32)) * S
    lt = jnp.swapaxes(logits, 0, 1)
    out_t = _sc_scale_t(lt, labels, ela_s)
    return jnp.swapaxes(out_t, 0, 1)


# speedup vs baseline: 4.2305x; 4.2305x over previous
import functools

import jax
import jax.numpy as jnp
from jax.experimental import pallas as pl

S = 64.0
MEAN = 0.35
SIGMA = 0.0125

N_ROWS = 1024
N_COLS = 100000
BR = 2048


def _body(lab_ref, ela_ref, lt_ref, out_ref):
    r0 = pl.program_id(0) * BR
    rows = r0 + jax.lax.broadcasted_iota(jnp.int32, (BR, N_ROWS), 0)
    hit = rows == lab_ref[:, :]
    out_ref[:, :] = lt_ref[:, :] * S - jnp.where(hit, ela_ref[:, :], 0.0)


def kernel(logits, labels):
    ekey = jax.random.key(42)
    ela_s = (MEAN + SIGMA * jax.random.normal(ekey, (N_ROWS,), dtype=jnp.float32)) * S
    lt = jnp.swapaxes(logits, 0, 1)
    lab2 = labels.reshape(1, N_ROWS)
    ela2 = ela_s.reshape(1, N_ROWS)

    grid = (pl.cdiv(N_COLS, BR),)
    out_t = pl.pallas_call(
        _body,
        grid=grid,
        in_specs=[
            pl.BlockSpec((1, N_ROWS), lambda i: (0, 0)),
            pl.BlockSpec((1, N_ROWS), lambda i: (0, 0)),
            pl.BlockSpec((BR, N_ROWS), lambda i: (i, 0)),
        ],
        out_specs=pl.BlockSpec((BR, N_ROWS), lambda i: (i, 0)),
        out_shape=jax.ShapeDtypeStruct((N_COLS, N_ROWS), jnp.float32),
    )(lab2, ela2, lt)
    return jnp.swapaxes(out_t, 0, 1)
